# baseline (speedup 1.0000x reference)
import math

import jax
import jax.numpy as jnp
from jax import lax
from jax.experimental import pallas as pl
from jax.experimental.pallas import tpu as pltpu
from jax.experimental.pallas import tpu_sc as plsc

_MAPN = 16777216
_MAPLEN = _MAPN + 16
_TRASH_S = _MAPN
_TRASH_G = _MAPN + 8
_NGT = 100000
_NQ = 50000
_NSRC = 30000
_GT_PT = 3200
_Q_PT = 1664


def _scatter_body(gr_ref, gs_ref, ov_ref, map_ref,
                  gr_v, gs_v, ov_v, sidx_v, ones_v, sem):
    i32 = jnp.int32
    f32 = jnp.float32
    c = lax.axis_index("c")
    s = lax.axis_index("s")
    w = c * 16 + s

    pltpu.sync_copy(gr_ref.at[pl.ds(w * _GT_PT, _GT_PT)], gr_v)
    pltpu.sync_copy(gs_ref.at[pl.ds(w * _GT_PT, _GT_PT)], gs_v)
    pltpu.sync_copy(ov_ref.at[pl.ds(w * _GT_PT, _GT_PT)], ov_v)

    ones16 = jnp.ones((16,), f32)
    for i in range(8):
        ones_v[pl.ds(i * 16, 16)] = ones16

    def _srow(r, carry):
        for j in range(8):
            off = r * 128 + j * 16
            key = gr_v[pl.ds(off, 16)] * 4096 + gs_v[pl.ds(off, 16)]
            ok = ov_v[pl.ds(off, 16)] > 0.1
            sidx_v[r, pl.ds(j * 16, 16)] = jnp.where(ok, key, _TRASH_S)
        return carry
    lax.fori_loop(0, 25, _srow, 0)

    copies = [pltpu.async_copy(ones_v, map_ref.at[sidx_v.at[r]], sem)
              for r in range(25)]
    for cp in copies:
        cp.wait()


def _gather_body(qr_ref, qs_ref, map_ref, part_ref,
                 qr_v, qs_v, gidx_v, gval_v, pacc_v, sem):
    i32 = jnp.int32
    f32 = jnp.float32
    c = lax.axis_index("c")
    s = lax.axis_index("s")
    w = c * 16 + s

    pltpu.sync_copy(qr_ref.at[pl.ds(w * _Q_PT, _Q_PT)], qr_v)
    pltpu.sync_copy(qs_ref.at[pl.ds(w * _Q_PT, _Q_PT)], qs_v)

    def _qrow(r, carry):
        for j in range(8):
            off = r * 128 + j * 16
            qk = qr_v[pl.ds(off, 16)] * 4096 + qs_v[pl.ds(off, 16)]
            gidx_v[r, pl.ds(j * 16, 16)] = qk
        return carry
    lax.fori_loop(0, 13, _qrow, 0)

    copies = [pltpu.async_copy(map_ref.at[gidx_v.at[r]], gval_v.at[r], sem)
              for r in range(13)]
    for cp in copies:
        cp.wait()

    def _acc(r, acc):
        for j in range(8):
            acc = acc + gval_v[r, pl.ds(j * 16, 16)]
        return acc
    acc = lax.fori_loop(0, 13, _acc, jnp.zeros((16,), f32))
    pacc_v[pl.ds(0, 16)] = acc
    pltpu.sync_copy(pacc_v, part_ref.at[w])


def _tc_body(rc_ref, scp_ref, sp_ref, tt_ref, et_ref, tb_ref, eb_ref,
             part_ref, out_ref):
    f32 = jnp.float32
    T = [[tt_ref[i, j] for j in range(4)] for i in range(4)]
    E = [[et_ref[i, j] for j in range(4)] for i in range(4)]

    cp = jnp.sum(part_ref[...]) * (1.0 / _NQ)

    sx = scp_ref[0:1, :]
    sy = scp_ref[1:2, :]
    sz = scp_ref[2:3, :]
    wx = T[0][0] * sx + T[0][1] * sy + T[0][2] * sz + T[0][3]
    wy = T[1][0] * sx + T[1][1] * sy + T[1][2] * sz + T[1][3]
    wz = T[2][0] * sx + T[2][1] * sy + T[2][2] * sz + T[2][3]
    dx = rc_ref[0:1, :] - wx
    dy = rc_ref[1:2, :] - wy
    dz = rc_ref[2:3, :] - wz
    d = jnp.sqrt(dx * dx + dy * dy + dz * dz)
    col = lax.broadcasted_iota(jnp.int32, d.shape, 1)
    fcnt = jnp.sum(jnp.where((d < 0.1) & (col < _NQ), 1.0, 0.0))
    f_prec = fcnt * (1.0 / _NQ)

    Tb = [[tb_ref[i, j] for j in range(4)] for i in range(4)]
    Eb = [[eb_ref[i, j] for j in range(4)] for i in range(4)]
    tr = sum(Tb[k][i] * Eb[k][i] for k in range(3) for i in range(3))
    x = jnp.clip(0.5 * (tr - 1.0), -1.0 + 1e-7, 1.0 - 1e-7)
    xa = jnp.full((8, 128), x, f32)
    a = jnp.abs(xa)
    p = (((((((-0.0012624911 * a + 0.0066700901) * a - 0.0170881256) * a
             + 0.0308918810) * a - 0.0501743046) * a + 0.0889789874) * a
          - 0.2145988016) * a + 1.5707963050)
    acv = jnp.sqrt(jnp.maximum(1.0 - a, 0.0)) * p
    acv = jnp.where(xa < 0.0, math.pi - acv, acv)
    rre_v = acv * (180.0 / math.pi)
    rte2 = ((T[0][3] - E[0][3]) ** 2 + (T[1][3] - E[1][3]) ** 2 +
            (T[2][3] - E[2][3]) ** 2)
    rte_v = jnp.sqrt(jnp.full((8, 128), rte2, f32))

    Rr = [[sum(T[k][i] * E[k][j] for k in range(3)) for j in range(3)]
          for i in range(3)]
    t_r = [sum(T[k][i] * (E[k][3] - T[k][3]) for k in range(3))
           for i in range(3)]
    px = sp_ref[0:1, :]
    py = sp_ref[1:2, :]
    pz = sp_ref[2:3, :]
    gx = Rr[0][0] * px + Rr[0][1] * py + Rr[0][2] * pz + t_r[0] - px
    gy = Rr[1][0] * px + Rr[1][1] * py + Rr[1][2] * pz + t_r[1] - py
    gz = Rr[2][0] * px + Rr[2][1] * py + Rr[2][2] * pz + t_r[2] - pz
    dn = jnp.sqrt(gx * gx + gy * gy + gz * gz)
    col2 = lax.broadcasted_iota(jnp.int32, dn.shape, 1)
    rmse = jnp.sum(jnp.where(col2 < _NSRC, dn, 0.0)) * (1.0 / _NSRC)
    recall = jnp.where(rmse < 0.2, 1.0, 0.0)

    lane = lax.broadcasted_iota(jnp.int32, (8, 128), 1)
    out = (jnp.where(lane == 0, cp, 0.0) +
           jnp.where(lane == 1, f_prec, 0.0) +
           jnp.where(lane == 2, rre_v, 0.0) +
           jnp.where(lane == 3, rte_v, 0.0) +
           jnp.where(lane == 4, rmse, 0.0) +
           jnp.where(lane == 5, recall, 0.0))
    out_ref[...] = out


def kernel(ref_points_c, src_points_c, gt_node_corr_overlaps,
           gt_node_corr_indices, ref_node_corr_indices,
           src_node_corr_indices, ref_corr_points, src_corr_points,
           src_points, transform, estimated_transform):
    i32 = jnp.int32
    f32 = jnp.float32
    ngt_p = 32 * _GT_PT
    nq_p = 32 * _Q_PT

    gr_p = jnp.pad(gt_node_corr_indices[:, 0].astype(i32), (0, ngt_p - _NGT))
    gs_p = jnp.pad(gt_node_corr_indices[:, 1].astype(i32), (0, ngt_p - _NGT))
    ov_p = jnp.pad(gt_node_corr_overlaps.astype(f32), (0, ngt_p - _NGT))
    qr_p = jnp.pad(ref_node_corr_indices.astype(i32), (0, nq_p - _NQ),
                   constant_values=4096)
    qs_p = jnp.pad(src_node_corr_indices.astype(i32), (0, nq_p - _NQ),
                   constant_values=8)

    mesh = plsc.VectorSubcoreMesh(core_axis_name="c", subcore_axis_name="s")
    map_ref = jax.new_ref(jnp.zeros((_MAPLEN,), f32))

    pl.kernel(
        _scatter_body,
        out_type=[],
        mesh=mesh,
        scratch_types=[
            pltpu.VMEM((_GT_PT,), i32),
            pltpu.VMEM((_GT_PT,), i32),
            pltpu.VMEM((_GT_PT,), f32),
            pltpu.VMEM((25, 128), i32),
            pltpu.VMEM((128,), f32),
            pltpu.SemaphoreType.DMA,
        ],
    )(gr_p, gs_p, ov_p, map_ref)

    partials = pl.kernel(
        _gather_body,
        out_type=jax.ShapeDtypeStruct((32, 16), f32),
        mesh=mesh,
        scratch_types=[
            pltpu.VMEM((_Q_PT,), i32),
            pltpu.VMEM((_Q_PT,), i32),
            pltpu.VMEM((13, 128), i32),
            pltpu.VMEM((13, 128), f32),
            pltpu.VMEM((16,), f32),
            pltpu.SemaphoreType.DMA,
        ],
    )(qr_p, qs_p, map_ref)

    nq_c = 50176
    ns_c = 30208
    rc_pad = jnp.zeros((8, nq_c), f32).at[:3, :_NQ].set(ref_corr_points.T)
    scp_pad = jnp.zeros((8, nq_c), f32).at[:3, :_NQ].set(src_corr_points.T)
    sp_pad = jnp.zeros((8, ns_c), f32).at[:3, :_NSRC].set(src_points.T)
    tf = transform.astype(f32)
    ef = estimated_transform.astype(f32)
    tb = tf.astype(jnp.bfloat16).astype(f32)
    eb = ef.astype(jnp.bfloat16).astype(f32)

    out = pl.pallas_call(
        _tc_body,
        out_shape=jax.ShapeDtypeStruct((8, 128), f32),
        in_specs=[
            pl.BlockSpec(memory_space=pltpu.VMEM),
            pl.BlockSpec(memory_space=pltpu.VMEM),
            pl.BlockSpec(memory_space=pltpu.VMEM),
            pl.BlockSpec(memory_space=pltpu.SMEM),
            pl.BlockSpec(memory_space=pltpu.SMEM),
            pl.BlockSpec(memory_space=pltpu.SMEM),
            pl.BlockSpec(memory_space=pltpu.SMEM),
            pl.BlockSpec(memory_space=pltpu.VMEM),
        ],
        out_specs=pl.BlockSpec(memory_space=pltpu.VMEM),
    )(rc_pad, scp_pad, sp_pad, tf, ef, tb, eb, partials)
    return out[0, 0:6]


# speedup vs baseline: 7.0452x; 7.0452x over previous
import math

import jax
import jax.numpy as jnp
from jax import lax
from jax.experimental import pallas as pl
from jax.experimental.pallas import tpu as pltpu
from jax.experimental.pallas import tpu_sc as plsc

_MAPN = 16777216
_MAPLEN = _MAPN + 2064
_TRASH_S = _MAPN
_TRASH_G = _MAPN + 2048 + 8
_NGT = 100000
_NQ = 50000
_NSRC = 30000
_GT_PT = 3200
_Q_PT = 1664


def _scatter_body(gr_ref, gs_ref, ov_ref, map_ref,
                  gr_v, gs_v, ov_v, sidx_v, ones_v, sem):
    i32 = jnp.int32
    f32 = jnp.float32
    c = lax.axis_index("c")
    s = lax.axis_index("s")
    w = c * 16 + s

    pltpu.sync_copy(gr_ref.at[pl.ds(w * _GT_PT, _GT_PT)], gr_v)
    pltpu.sync_copy(gs_ref.at[pl.ds(w * _GT_PT, _GT_PT)], gs_v)
    pltpu.sync_copy(ov_ref.at[pl.ds(w * _GT_PT, _GT_PT)], ov_v)

    ones16 = jnp.ones((16,), f32)
    for i in range(8):
        ones_v[pl.ds(i * 16, 16)] = ones16

    lanei = lax.iota(i32, 16)

    def _srow(r, carry):
        for j in range(8):
            off = r * 128 + j * 16
            key = gr_v[pl.ds(off, 16)] * 4096 + gs_v[pl.ds(off, 16)]
            ok = ov_v[pl.ds(off, 16)] > 0.1
            trash = _TRASH_S + ((off + lanei) & 2047)
            sidx_v[r, pl.ds(j * 16, 16)] = jnp.where(ok, key, trash)
        return carry
    lax.fori_loop(0, 25, _srow, 0)

    copies = [pltpu.async_copy(ones_v, map_ref.at[sidx_v.at[r]], sem)
              for r in range(25)]
    for cp in copies:
        cp.wait()


def _gather_body(qr_ref, qs_ref, map_ref, part_ref,
                 qr_v, qs_v, gidx_v, gval_v, pacc_v, sem):
    i32 = jnp.int32
    f32 = jnp.float32
    c = lax.axis_index("c")
    s = lax.axis_index("s")
    w = c * 16 + s

    pltpu.sync_copy(qr_ref.at[pl.ds(w * _Q_PT, _Q_PT)], qr_v)
    pltpu.sync_copy(qs_ref.at[pl.ds(w * _Q_PT, _Q_PT)], qs_v)

    def _qrow(r, carry):
        for j in range(8):
            off = r * 128 + j * 16
            qk = qr_v[pl.ds(off, 16)] * 4096 + qs_v[pl.ds(off, 16)]
            gidx_v[r, pl.ds(j * 16, 16)] = qk
        return carry
    lax.fori_loop(0, 13, _qrow, 0)

    copies = [pltpu.async_copy(map_ref.at[gidx_v.at[r]], gval_v.at[r], sem)
              for r in range(13)]
    for cp in copies:
        cp.wait()

    def _acc(r, acc):
        for j in range(8):
            acc = acc + gval_v[r, pl.ds(j * 16, 16)]
        return acc
    acc = lax.fori_loop(0, 13, _acc, jnp.zeros((16,), f32))
    pacc_v[pl.ds(0, 16)] = acc
    pltpu.sync_copy(pacc_v, part_ref.at[w])


def _tc_body(rc_ref, scp_ref, sp_ref, tt_ref, et_ref, tb_ref, eb_ref,
             part_ref, out_ref):
    f32 = jnp.float32
    T = [[tt_ref[i, j] for j in range(4)] for i in range(4)]
    E = [[et_ref[i, j] for j in range(4)] for i in range(4)]

    cp = jnp.sum(part_ref[...]) * (1.0 / _NQ)

    sx = scp_ref[0:1, :]
    sy = scp_ref[1:2, :]
    sz = scp_ref[2:3, :]
    wx = T[0][0] * sx + T[0][1] * sy + T[0][2] * sz + T[0][3]
    wy = T[1][0] * sx + T[1][1] * sy + T[1][2] * sz + T[1][3]
    wz = T[2][0] * sx + T[2][1] * sy + T[2][2] * sz + T[2][3]
    dx = rc_ref[0:1, :] - wx
    dy = rc_ref[1:2, :] - wy
    dz = rc_ref[2:3, :] - wz
    d = jnp.sqrt(dx * dx + dy * dy + dz * dz)
    col = lax.broadcasted_iota(jnp.int32, d.shape, 1)
    fcnt = jnp.sum(jnp.where((d < 0.1) & (col < _NQ), 1.0, 0.0))
    f_prec = fcnt * (1.0 / _NQ)

    Tb = [[tb_ref[i, j] for j in range(4)] for i in range(4)]
    Eb = [[eb_ref[i, j] for j in range(4)] for i in range(4)]
    tr = sum(Tb[k][i] * Eb[k][i] for k in range(3) for i in range(3))
    x = jnp.clip(0.5 * (tr - 1.0), -1.0 + 1e-7, 1.0 - 1e-7)
    xa = jnp.full((8, 128), x, f32)
    a = jnp.abs(xa)
    p = (((((((-0.0012624911 * a + 0.0066700901) * a - 0.0170881256) * a
             + 0.0308918810) * a - 0.0501743046) * a + 0.0889789874) * a
          - 0.2145988016) * a + 1.5707963050)
    acv = jnp.sqrt(jnp.maximum(1.0 - a, 0.0)) * p
    acv = jnp.where(xa < 0.0, math.pi - acv, acv)
    rre_v = acv * (180.0 / math.pi)
    rte2 = ((T[0][3] - E[0][3]) ** 2 + (T[1][3] - E[1][3]) ** 2 +
            (T[2][3] - E[2][3]) ** 2)
    rte_v = jnp.sqrt(jnp.full((8, 128), rte2, f32))

    Rr = [[sum(T[k][i] * E[k][j] for k in range(3)) for j in range(3)]
          for i in range(3)]
    t_r = [sum(T[k][i] * (E[k][3] - T[k][3]) for k in range(3))
           for i in range(3)]
    px = sp_ref[0:1, :]
    py = sp_ref[1:2, :]
    pz = sp_ref[2:3, :]
    gx = Rr[0][0] * px + Rr[0][1] * py + Rr[0][2] * pz + t_r[0] - px
    gy = Rr[1][0] * px + Rr[1][1] * py + Rr[1][2] * pz + t_r[1] - py
    gz = Rr[2][0] * px + Rr[2][1] * py + Rr[2][2] * pz + t_r[2] - pz
    dn = jnp.sqrt(gx * gx + gy * gy + gz * gz)
    col2 = lax.broadcasted_iota(jnp.int32, dn.shape, 1)
    rmse = jnp.sum(jnp.where(col2 < _NSRC, dn, 0.0)) * (1.0 / _NSRC)
    recall = jnp.where(rmse < 0.2, 1.0, 0.0)

    lane = lax.broadcasted_iota(jnp.int32, (8, 128), 1)
    out = (jnp.where(lane == 0, cp, 0.0) +
           jnp.where(lane == 1, f_prec, 0.0) +
           jnp.where(lane == 2, rre_v, 0.0) +
           jnp.where(lane == 3, rte_v, 0.0) +
           jnp.where(lane == 4, rmse, 0.0) +
           jnp.where(lane == 5, recall, 0.0))
    out_ref[...] = out


def kernel(ref_points_c, src_points_c, gt_node_corr_overlaps,
           gt_node_corr_indices, ref_node_corr_indices,
           src_node_corr_indices, ref_corr_points, src_corr_points,
           src_points, transform, estimated_transform):
    i32 = jnp.int32
    f32 = jnp.float32
    ngt_p = 32 * _GT_PT
    nq_p = 32 * _Q_PT

    gr_p = jnp.pad(gt_node_corr_indices[:, 0].astype(i32), (0, ngt_p - _NGT))
    gs_p = jnp.pad(gt_node_corr_indices[:, 1].astype(i32), (0, ngt_p - _NGT))
    ov_p = jnp.pad(gt_node_corr_overlaps.astype(f32), (0, ngt_p - _NGT))
    qr_p = jnp.pad(ref_node_corr_indices.astype(i32), (0, nq_p - _NQ),
                   constant_values=4096)
    qs_p = jnp.pad(src_node_corr_indices.astype(i32), (0, nq_p - _NQ),
                   constant_values=_TRASH_G - _MAPN)

    mesh = plsc.VectorSubcoreMesh(core_axis_name="c", subcore_axis_name="s")
    map_ref = jax.new_ref(jnp.zeros((_MAPLEN,), f32))

    pl.kernel(
        _scatter_body,
        out_type=[],
        mesh=mesh,
        scratch_types=[
            pltpu.VMEM((_GT_PT,), i32),
            pltpu.VMEM((_GT_PT,), i32),
            pltpu.VMEM((_GT_PT,), f32),
            pltpu.VMEM((25, 128), i32),
            pltpu.VMEM((128,), f32),
            pltpu.SemaphoreType.DMA,
        ],
    )(gr_p, gs_p, ov_p, map_ref)

    partials = pl.kernel(
        _gather_body,
        out_type=jax.ShapeDtypeStruct((32, 16), f32),
        mesh=mesh,
        scratch_types=[
            pltpu.VMEM((_Q_PT,), i32),
            pltpu.VMEM((_Q_PT,), i32),
            pltpu.VMEM((13, 128), i32),
            pltpu.VMEM((13, 128), f32),
            pltpu.VMEM((16,), f32),
            pltpu.SemaphoreType.DMA,
        ],
    )(qr_p, qs_p, map_ref)

    nq_c = 50176
    ns_c = 30208
    rc_pad = jnp.zeros((8, nq_c), f32).at[:3, :_NQ].set(ref_corr_points.T)
    scp_pad = jnp.zeros((8, nq_c), f32).at[:3, :_NQ].set(src_corr_points.T)
    sp_pad = jnp.zeros((8, ns_c), f32).at[:3, :_NSRC].set(src_points.T)
    tf = transform.astype(f32)
    ef = estimated_transform.astype(f32)
    tb = tf.astype(jnp.bfloat16).astype(f32)
    eb = ef.astype(jnp.bfloat16).astype(f32)

    out = pl.pallas_call(
        _tc_body,
        out_shape=jax.ShapeDtypeStruct((8, 128), f32),
        in_specs=[
            pl.BlockSpec(memory_space=pltpu.VMEM),
            pl.BlockSpec(memory_space=pltpu.VMEM),
            pl.BlockSpec(memory_space=pltpu.VMEM),
            pl.BlockSpec(memory_space=pltpu.SMEM),
            pl.BlockSpec(memory_space=pltpu.SMEM),
            pl.BlockSpec(memory_space=pltpu.SMEM),
            pl.BlockSpec(memory_space=pltpu.SMEM),
            pl.BlockSpec(memory_space=pltpu.VMEM),
        ],
        out_specs=pl.BlockSpec(memory_space=pltpu.VMEM),
    )(rc_pad, scp_pad, sp_pad, tf, ef, tb, eb, partials)
    return out[0, 0:6]


# speedup vs baseline: 10.1094x; 1.4349x over previous
import math

import jax
import jax.numpy as jnp
from jax import lax
from jax.experimental import pallas as pl
from jax.experimental.pallas import tpu as pltpu
from jax.experimental.pallas import tpu_sc as plsc

_MAPN = 16777216
_MAPLEN = _MAPN + 65552
_TRASH_S = _MAPN
_TRASH_G = _MAPN + 65536 + 8
_NGT = 100000
_NQ = 50000
_NSRC = 30000
_GT_PT = 3200
_Q_PT = 1664


def _scatter_body(gr_ref, gs_ref, ov_ref, map_ref,
                  gr_v, gs_v, ov_v, sidx_v, ones_v, sem):
    i32 = jnp.int32
    f32 = jnp.float32
    c = lax.axis_index("c")
    s = lax.axis_index("s")
    w = c * 16 + s

    pltpu.sync_copy(gr_ref.at[pl.ds(w * _GT_PT, _GT_PT)], gr_v)
    pltpu.sync_copy(gs_ref.at[pl.ds(w * _GT_PT, _GT_PT)], gs_v)
    pltpu.sync_copy(ov_ref.at[pl.ds(w * _GT_PT, _GT_PT)], ov_v)

    ones16 = jnp.ones((16,), f32)
    for i in range(8):
        ones_v[pl.ds(i * 16, 16)] = ones16

    lanei = lax.iota(i32, 16)

    def _srow(r, carry):
        for j in range(8):
            off = r * 128 + j * 16
            key = gr_v[pl.ds(off, 16)] * 4096 + gs_v[pl.ds(off, 16)]
            ok = ov_v[pl.ds(off, 16)] > 0.1
            trash = _TRASH_S + w * 2048 + ((off + lanei) & 2047)
            sidx_v[r, pl.ds(j * 16, 16)] = jnp.where(ok, key, trash)
        return carry
    lax.fori_loop(0, 25, _srow, 0)

    copies = [pltpu.async_copy(ones_v, map_ref.at[sidx_v.at[r]], sem)
              for r in range(25)]
    for cp in copies:
        cp.wait()


def _gather_body(qr_ref, qs_ref, map_ref, part_ref,
                 qr_v, qs_v, gidx_v, gval_v, pacc_v, sem):
    i32 = jnp.int32
    f32 = jnp.float32
    c = lax.axis_index("c")
    s = lax.axis_index("s")
    w = c * 16 + s

    pltpu.sync_copy(qr_ref.at[pl.ds(w * _Q_PT, _Q_PT)], qr_v)
    pltpu.sync_copy(qs_ref.at[pl.ds(w * _Q_PT, _Q_PT)], qs_v)

    def _qrow(r, carry):
        for j in range(8):
            off = r * 128 + j * 16
            qk = qr_v[pl.ds(off, 16)] * 4096 + qs_v[pl.ds(off, 16)]
            gidx_v[r, pl.ds(j * 16, 16)] = qk
        return carry
    lax.fori_loop(0, 13, _qrow, 0)

    copies = [pltpu.async_copy(map_ref.at[gidx_v.at[r]], gval_v.at[r], sem)
              for r in range(13)]
    for cp in copies:
        cp.wait()

    def _acc(r, acc):
        for j in range(8):
            acc = acc + gval_v[r, pl.ds(j * 16, 16)]
        return acc
    acc = lax.fori_loop(0, 13, _acc, jnp.zeros((16,), f32))
    pacc_v[pl.ds(0, 16)] = acc
    pltpu.sync_copy(pacc_v, part_ref.at[w])


def _tc_body(rc_ref, scp_ref, sp_ref, tt_ref, et_ref, tb_ref, eb_ref,
             part_ref, out_ref):
    f32 = jnp.float32
    T = [[tt_ref[i, j] for j in range(4)] for i in range(4)]
    E = [[et_ref[i, j] for j in range(4)] for i in range(4)]

    cp = jnp.sum(part_ref[...]) * (1.0 / _NQ)

    sx = scp_ref[0:1, :]
    sy = scp_ref[1:2, :]
    sz = scp_ref[2:3, :]
    wx = T[0][0] * sx + T[0][1] * sy + T[0][2] * sz + T[0][3]
    wy = T[1][0] * sx + T[1][1] * sy + T[1][2] * sz + T[1][3]
    wz = T[2][0] * sx + T[2][1] * sy + T[2][2] * sz + T[2][3]
    dx = rc_ref[0:1, :] - wx
    dy = rc_ref[1:2, :] - wy
    dz = rc_ref[2:3, :] - wz
    d = jnp.sqrt(dx * dx + dy * dy + dz * dz)
    col = lax.broadcasted_iota(jnp.int32, d.shape, 1)
    fcnt = jnp.sum(jnp.where((d < 0.1) & (col < _NQ), 1.0, 0.0))
    f_prec = fcnt * (1.0 / _NQ)

    Tb = [[tb_ref[i, j] for j in range(4)] for i in range(4)]
    Eb = [[eb_ref[i, j] for j in range(4)] for i in range(4)]
    tr = sum(Tb[k][i] * Eb[k][i] for k in range(3) for i in range(3))
    x = jnp.clip(0.5 * (tr - 1.0), -1.0 + 1e-7, 1.0 - 1e-7)
    xa = jnp.full((8, 128), x, f32)
    a = jnp.abs(xa)
    p = (((((((-0.0012624911 * a + 0.0066700901) * a - 0.0170881256) * a
             + 0.0308918810) * a - 0.0501743046) * a + 0.0889789874) * a
          - 0.2145988016) * a + 1.5707963050)
    acv = jnp.sqrt(jnp.maximum(1.0 - a, 0.0)) * p
    acv = jnp.where(xa < 0.0, math.pi - acv, acv)
    rre_v = acv * (180.0 / math.pi)
    rte2 = ((T[0][3] - E[0][3]) ** 2 + (T[1][3] - E[1][3]) ** 2 +
            (T[2][3] - E[2][3]) ** 2)
    rte_v = jnp.sqrt(jnp.full((8, 128), rte2, f32))

    Rr = [[sum(T[k][i] * E[k][j] for k in range(3)) for j in range(3)]
          for i in range(3)]
    t_r = [sum(T[k][i] * (E[k][3] - T[k][3]) for k in range(3))
           for i in range(3)]
    px = sp_ref[0:1, :]
    py = sp_ref[1:2, :]
    pz = sp_ref[2:3, :]
    gx = Rr[0][0] * px + Rr[0][1] * py + Rr[0][2] * pz + t_r[0] - px
    gy = Rr[1][0] * px + Rr[1][1] * py + Rr[1][2] * pz + t_r[1] - py
    gz = Rr[2][0] * px + Rr[2][1] * py + Rr[2][2] * pz + t_r[2] - pz
    dn = jnp.sqrt(gx * gx + gy * gy + gz * gz)
    col2 = lax.broadcasted_iota(jnp.int32, dn.shape, 1)
    rmse = jnp.sum(jnp.where(col2 < _NSRC, dn, 0.0)) * (1.0 / _NSRC)
    recall = jnp.where(rmse < 0.2, 1.0, 0.0)

    lane = lax.broadcasted_iota(jnp.int32, (8, 128), 1)
    out = (jnp.where(lane == 0, cp, 0.0) +
           jnp.where(lane == 1, f_prec, 0.0) +
           jnp.where(lane == 2, rre_v, 0.0) +
           jnp.where(lane == 3, rte_v, 0.0) +
           jnp.where(lane == 4, rmse, 0.0) +
           jnp.where(lane == 5, recall, 0.0))
    out_ref[...] = out


def kernel(ref_points_c, src_points_c, gt_node_corr_overlaps,
           gt_node_corr_indices, ref_node_corr_indices,
           src_node_corr_indices, ref_corr_points, src_corr_points,
           src_points, transform, estimated_transform):
    i32 = jnp.int32
    f32 = jnp.float32
    ngt_p = 32 * _GT_PT
    nq_p = 32 * _Q_PT

    gr_p = jnp.pad(gt_node_corr_indices[:, 0].astype(i32), (0, ngt_p - _NGT))
    gs_p = jnp.pad(gt_node_corr_indices[:, 1].astype(i32), (0, ngt_p - _NGT))
    ov_p = jnp.pad(gt_node_corr_overlaps.astype(f32), (0, ngt_p - _NGT))
    qr_p = jnp.pad(ref_node_corr_indices.astype(i32), (0, nq_p - _NQ),
                   constant_values=4096)
    qs_p = jnp.pad(src_node_corr_indices.astype(i32), (0, nq_p - _NQ),
                   constant_values=_TRASH_G - _MAPN)

    mesh = plsc.VectorSubcoreMesh(core_axis_name="c", subcore_axis_name="s")
    map_ref = jax.new_ref(jnp.zeros((_MAPLEN,), f32))

    pl.kernel(
        _scatter_body,
        out_type=[],
        mesh=mesh,
        scratch_types=[
            pltpu.VMEM((_GT_PT,), i32),
            pltpu.VMEM((_GT_PT,), i32),
            pltpu.VMEM((_GT_PT,), f32),
            pltpu.VMEM((25, 128), i32),
            pltpu.VMEM((128,), f32),
            pltpu.SemaphoreType.DMA,
        ],
    )(gr_p, gs_p, ov_p, map_ref)

    partials = pl.kernel(
        _gather_body,
        out_type=jax.ShapeDtypeStruct((32, 16), f32),
        mesh=mesh,
        scratch_types=[
            pltpu.VMEM((_Q_PT,), i32),
            pltpu.VMEM((_Q_PT,), i32),
            pltpu.VMEM((13, 128), i32),
            pltpu.VMEM((13, 128), f32),
            pltpu.VMEM((16,), f32),
            pltpu.SemaphoreType.DMA,
        ],
    )(qr_p, qs_p, map_ref)

    nq_c = 50176
    ns_c = 30208
    rc_pad = jnp.zeros((8, nq_c), f32).at[:3, :_NQ].set(ref_corr_points.T)
    scp_pad = jnp.zeros((8, nq_c), f32).at[:3, :_NQ].set(src_corr_points.T)
    sp_pad = jnp.zeros((8, ns_c), f32).at[:3, :_NSRC].set(src_points.T)
    tf = transform.astype(f32)
    ef = estimated_transform.astype(f32)
    tb = tf.astype(jnp.bfloat16).astype(f32)
    eb = ef.astype(jnp.bfloat16).astype(f32)

    out = pl.pallas_call(
        _tc_body,
        out_shape=jax.ShapeDtypeStruct((8, 128), f32),
        in_specs=[
            pl.BlockSpec(memory_space=pltpu.VMEM),
            pl.BlockSpec(memory_space=pltpu.VMEM),
            pl.BlockSpec(memory_space=pltpu.VMEM),
            pl.BlockSpec(memory_space=pltpu.SMEM),
            pl.BlockSpec(memory_space=pltpu.SMEM),
            pl.BlockSpec(memory_space=pltpu.SMEM),
            pl.BlockSpec(memory_space=pltpu.SMEM),
            pl.BlockSpec(memory_space=pltpu.VMEM),
        ],
        out_specs=pl.BlockSpec(memory_space=pltpu.VMEM),
    )(rc_pad, scp_pad, sp_pad, tf, ef, tb, eb, partials)
    return out[0, 0:6]


# speedup vs baseline: 10.1105x; 1.0001x over previous
import math

import jax
import jax.numpy as jnp
from jax import lax
from jax.experimental import pallas as pl
from jax.experimental.pallas import tpu as pltpu
from jax.experimental.pallas import tpu_sc as plsc

_MAPN = 16777216
_MAPLEN = _MAPN + 65552
_TRASH_S = _MAPN
_TRASH_G = _MAPN + 65536 + 8
_NGT = 100000
_NQ = 50000
_NSRC = 30000
_GT_PT = 3200
_Q_PT = 1664


def _scatter_body(gr_ref, gs_ref, ov_ref, map_ref,
                  gr_v, gs_v, ov_v, sidx_v, ones_v, sem):
    i32 = jnp.int32
    f32 = jnp.float32
    c = lax.axis_index("c")
    s = lax.axis_index("s")
    w = c * 16 + s

    pltpu.sync_copy(gr_ref.at[pl.ds(w * _GT_PT, _GT_PT)], gr_v)
    pltpu.sync_copy(gs_ref.at[pl.ds(w * _GT_PT, _GT_PT)], gs_v)
    pltpu.sync_copy(ov_ref.at[pl.ds(w * _GT_PT, _GT_PT)], ov_v)

    ones16 = jnp.ones((16,), f32)
    for i in range(8):
        ones_v[pl.ds(i * 16, 16)] = ones16

    lanei = lax.iota(i32, 16)

    def _srow(r, carry):
        for j in range(8):
            off = r * 128 + j * 16
            key = gr_v[pl.ds(off, 16)] * 4096 + gs_v[pl.ds(off, 16)]
            ok = ov_v[pl.ds(off, 16)] > 0.1
            trash = _TRASH_S + w * 2048 + ((off + lanei) & 2047)
            sidx_v[r, pl.ds(j * 16, 16)] = jnp.where(ok, key, trash)
        return carry
    lax.fori_loop(0, 25, _srow, 0)

    copies = [pltpu.async_copy(ones_v, map_ref.at[sidx_v.at[r]], sem)
              for r in range(25)]
    for cp in copies:
        cp.wait()


def _gather_body(qr_ref, qs_ref, map_ref, part_ref,
                 qr_v, qs_v, gidx_v, gval_v, pacc_v, sem):
    i32 = jnp.int32
    f32 = jnp.float32
    c = lax.axis_index("c")
    s = lax.axis_index("s")
    w = c * 16 + s

    pltpu.sync_copy(qr_ref.at[pl.ds(w * _Q_PT, _Q_PT)], qr_v)
    pltpu.sync_copy(qs_ref.at[pl.ds(w * _Q_PT, _Q_PT)], qs_v)

    def _qrow(r, carry):
        for j in range(8):
            off = r * 128 + j * 16
            qk = qr_v[pl.ds(off, 16)] * 4096 + qs_v[pl.ds(off, 16)]
            gidx_v[r, pl.ds(j * 16, 16)] = qk
        return carry
    lax.fori_loop(0, 13, _qrow, 0)

    copies = [pltpu.async_copy(map_ref.at[gidx_v.at[r]], gval_v.at[r], sem)
              for r in range(13)]
    for cp in copies:
        cp.wait()

    def _acc(r, acc):
        for j in range(8):
            acc = acc + gval_v[r, pl.ds(j * 16, 16)]
        return acc
    acc = lax.fori_loop(0, 13, _acc, jnp.zeros((16,), f32))
    pacc_v[pl.ds(0, 16)] = acc
    pltpu.sync_copy(pacc_v, part_ref.at[w])


def _tc_body(rc_ref, scp_ref, sp_ref, tt_ref, et_ref, tb_ref, eb_ref,
             part_ref, out_ref):
    f32 = jnp.float32
    T = [[tt_ref[i, j] for j in range(4)] for i in range(4)]
    E = [[et_ref[i, j] for j in range(4)] for i in range(4)]

    cp = jnp.sum(part_ref[...]) * (1.0 / _NQ)

    sx = scp_ref[0:1, :]
    sy = scp_ref[1:2, :]
    sz = scp_ref[2:3, :]
    wx = T[0][0] * sx + T[0][1] * sy + T[0][2] * sz + T[0][3]
    wy = T[1][0] * sx + T[1][1] * sy + T[1][2] * sz + T[1][3]
    wz = T[2][0] * sx + T[2][1] * sy + T[2][2] * sz + T[2][3]
    dx = rc_ref[0:1, :] - wx
    dy = rc_ref[1:2, :] - wy
    dz = rc_ref[2:3, :] - wz
    d = jnp.sqrt(dx * dx + dy * dy + dz * dz)
    col = lax.broadcasted_iota(jnp.int32, d.shape, 1)
    fcnt = jnp.sum(jnp.where((d < 0.1) & (col < _NQ), 1.0, 0.0))
    f_prec = fcnt * (1.0 / _NQ)

    Tb = [[tb_ref[i, j] for j in range(4)] for i in range(4)]
    Eb = [[eb_ref[i, j] for j in range(4)] for i in range(4)]
    tr = sum(Tb[k][i] * Eb[k][i] for k in range(3) for i in range(3))
    x = jnp.clip(0.5 * (tr - 1.0), -1.0 + 1e-7, 1.0 - 1e-7)
    xa = jnp.full((8, 128), x, f32)
    a = jnp.abs(xa)
    p = (((((((-0.0012624911 * a + 0.0066700901) * a - 0.0170881256) * a
             + 0.0308918810) * a - 0.0501743046) * a + 0.0889789874) * a
          - 0.2145988016) * a + 1.5707963050)
    acv = jnp.sqrt(jnp.maximum(1.0 - a, 0.0)) * p
    acv = jnp.where(xa < 0.0, math.pi - acv, acv)
    rre_v = acv * (180.0 / math.pi)
    rte2 = ((T[0][3] - E[0][3]) ** 2 + (T[1][3] - E[1][3]) ** 2 +
            (T[2][3] - E[2][3]) ** 2)
    rte_v = jnp.sqrt(jnp.full((8, 128), rte2, f32))

    Rr = [[sum(T[k][i] * E[k][j] for k in range(3)) for j in range(3)]
          for i in range(3)]
    t_r = [sum(T[k][i] * (E[k][3] - T[k][3]) for k in range(3))
           for i in range(3)]
    px = sp_ref[0:1, :]
    py = sp_ref[1:2, :]
    pz = sp_ref[2:3, :]
    gx = Rr[0][0] * px + Rr[0][1] * py + Rr[0][2] * pz + t_r[0] - px
    gy = Rr[1][0] * px + Rr[1][1] * py + Rr[1][2] * pz + t_r[1] - py
    gz = Rr[2][0] * px + Rr[2][1] * py + Rr[2][2] * pz + t_r[2] - pz
    dn = jnp.sqrt(gx * gx + gy * gy + gz * gz)
    col2 = lax.broadcasted_iota(jnp.int32, dn.shape, 1)
    rmse = jnp.sum(jnp.where(col2 < _NSRC, dn, 0.0)) * (1.0 / _NSRC)
    recall = jnp.where(rmse < 0.2, 1.0, 0.0)

    lane = lax.broadcasted_iota(jnp.int32, (8, 128), 1)
    out = (jnp.where(lane == 0, cp, 0.0) +
           jnp.where(lane == 1, f_prec, 0.0) +
           jnp.where(lane == 2, rre_v, 0.0) +
           jnp.where(lane == 3, rte_v, 0.0) +
           jnp.where(lane == 4, rmse, 0.0) +
           jnp.where(lane == 5, recall, 0.0))
    out_ref[...] = out


def kernel(ref_points_c, src_points_c, gt_node_corr_overlaps,
           gt_node_corr_indices, ref_node_corr_indices,
           src_node_corr_indices, ref_corr_points, src_corr_points,
           src_points, transform, estimated_transform):
    i32 = jnp.int32
    f32 = jnp.float32
    ngt_p = 32 * _GT_PT
    nq_p = 32 * _Q_PT

    gr_p = jnp.pad(gt_node_corr_indices[:, 0].astype(i32), (0, ngt_p - _NGT))
    gs_p = jnp.pad(gt_node_corr_indices[:, 1].astype(i32), (0, ngt_p - _NGT))
    ov_p = jnp.pad(gt_node_corr_overlaps.astype(f32), (0, ngt_p - _NGT))
    qr_p = jnp.pad(ref_node_corr_indices.astype(i32), (0, nq_p - _NQ),
                   constant_values=4096)
    qs_p = jnp.pad(src_node_corr_indices.astype(i32), (0, nq_p - _NQ),
                   constant_values=_TRASH_G - _MAPN)

    mesh = plsc.VectorSubcoreMesh(core_axis_name="c", subcore_axis_name="s")
    map_ref = jax.new_ref(jnp.zeros((_MAPLEN,), f32))

    pl.kernel(
        _scatter_body,
        out_type=[],
        mesh=mesh,
        scratch_types=[
            pltpu.VMEM((_GT_PT,), i32),
            pltpu.VMEM((_GT_PT,), i32),
            pltpu.VMEM((_GT_PT,), f32),
            pltpu.VMEM((25, 128), i32),
            pltpu.VMEM((128,), f32),
            pltpu.SemaphoreType.DMA,
        ],
    )(gr_p, gs_p, ov_p, map_ref)

    partials = pl.kernel(
        _gather_body,
        out_type=jax.ShapeDtypeStruct((32, 16), f32),
        mesh=mesh,
        scratch_types=[
            pltpu.VMEM((_Q_PT,), i32),
            pltpu.VMEM((_Q_PT,), i32),
            pltpu.VMEM((13, 128), i32),
            pltpu.VMEM((13, 128), f32),
            pltpu.VMEM((16,), f32),
            pltpu.SemaphoreType.DMA,
        ],
    )(qr_p, qs_p, map_ref)

    nq_c = 50176
    ns_c = 30208
    rc_pad = jnp.zeros((8, nq_c), f32).at[:3, :_NQ].set(ref_corr_points.T)
    scp_pad = jnp.zeros((8, nq_c), f32).at[:3, :_NQ].set(src_corr_points.T)
    sp_pad = jnp.zeros((8, ns_c), f32).at[:3, :_NSRC].set(src_points.T)
    tf = transform.astype(f32)
    ef = estimated_transform.astype(f32)

    def _round_bf16(x):
        xi = lax.bitcast_convert_type(x, i32)
        r = (xi + 0x7FFF + ((xi >> 16) & 1)) & (-65536)
        return lax.bitcast_convert_type(r, f32)

    tb = _round_bf16(tf)
    eb = _round_bf16(ef)

    out = pl.pallas_call(
        _tc_body,
        out_shape=jax.ShapeDtypeStruct((8, 128), f32),
        in_specs=[
            pl.BlockSpec(memory_space=pltpu.VMEM),
            pl.BlockSpec(memory_space=pltpu.VMEM),
            pl.BlockSpec(memory_space=pltpu.VMEM),
            pl.BlockSpec(memory_space=pltpu.SMEM),
            pl.BlockSpec(memory_space=pltpu.SMEM),
            pl.BlockSpec(memory_space=pltpu.SMEM),
            pl.BlockSpec(memory_space=pltpu.SMEM),
            pl.BlockSpec(memory_space=pltpu.VMEM),
        ],
        out_specs=pl.BlockSpec(memory_space=pltpu.VMEM),
    )(rc_pad, scp_pad, sp_pad, tf, ef, tb, eb, partials)
    return out[0, 0:6]


# speedup vs baseline: 10.2295x; 1.0118x over previous
import math

import jax
import jax.numpy as jnp
from jax import lax
from jax.experimental import pallas as pl
from jax.experimental.pallas import tpu as pltpu
from jax.experimental.pallas import tpu_sc as plsc

_MAPN = 16777216
_MAPLEN = _MAPN + 65552
_TRASH_S = _MAPN
_TRASH_G = _MAPN + 65536 + 8
_NGT = 100000
_NQ = 50000
_NSRC = 30000
_GT_PT = 3200
_Q_PT = 1664


def _scatter_body(gr_ref, gs_ref, ov_ref, map_ref,
                  gr_v, gs_v, ov_v, sidx_v, ones_v, sem, sem2, sem3, sem4):
    i32 = jnp.int32
    f32 = jnp.float32
    c = lax.axis_index("c")
    s = lax.axis_index("s")
    w = c * 16 + s

    pltpu.sync_copy(gr_ref.at[pl.ds(w * _GT_PT, _GT_PT)], gr_v)
    pltpu.sync_copy(gs_ref.at[pl.ds(w * _GT_PT, _GT_PT)], gs_v)
    pltpu.sync_copy(ov_ref.at[pl.ds(w * _GT_PT, _GT_PT)], ov_v)

    ones16 = jnp.ones((16,), f32)
    for i in range(8):
        ones_v[pl.ds(i * 16, 16)] = ones16

    lanei = lax.iota(i32, 16)

    def _srow(r, carry):
        for j in range(8):
            off = r * 128 + j * 16
            key = gr_v[pl.ds(off, 16)] * 4096 + gs_v[pl.ds(off, 16)]
            ok = ov_v[pl.ds(off, 16)] > 0.1
            trash = _TRASH_S + w * 2048 + ((off + lanei) & 2047)
            sidx_v[r, pl.ds(j * 16, 16)] = jnp.where(ok, key, trash)
        return carry
    lax.fori_loop(0, 25, _srow, 0)

    sems = [sem, sem2, sem3, sem4]
    copies = [pltpu.async_copy(ones_v, map_ref.at[sidx_v.at[r]], sems[r % 4])
              for r in range(25)]
    for cp in copies:
        cp.wait()


def _gather_body(qr_ref, qs_ref, map_ref, part_ref,
                 qr_v, qs_v, gidx_v, gval_v, pacc_v, sem, sem2):
    i32 = jnp.int32
    f32 = jnp.float32
    c = lax.axis_index("c")
    s = lax.axis_index("s")
    w = c * 16 + s

    pltpu.sync_copy(qr_ref.at[pl.ds(w * _Q_PT, _Q_PT)], qr_v)
    pltpu.sync_copy(qs_ref.at[pl.ds(w * _Q_PT, _Q_PT)], qs_v)

    def _qrow(r, carry):
        for j in range(8):
            off = r * 128 + j * 16
            qk = qr_v[pl.ds(off, 16)] * 4096 + qs_v[pl.ds(off, 16)]
            gidx_v[r, pl.ds(j * 16, 16)] = qk
        return carry
    lax.fori_loop(0, 13, _qrow, 0)

    gsems = [sem, sem2]
    copies = [pltpu.async_copy(map_ref.at[gidx_v.at[r]], gval_v.at[r],
                               gsems[r % 2])
              for r in range(13)]
    for cp in copies:
        cp.wait()

    def _acc(r, acc):
        for j in range(8):
            acc = acc + gval_v[r, pl.ds(j * 16, 16)]
        return acc
    acc = lax.fori_loop(0, 13, _acc, jnp.zeros((16,), f32))
    pacc_v[pl.ds(0, 16)] = acc
    pltpu.sync_copy(pacc_v, part_ref.at[w])


def _tc_body(rc_ref, scp_ref, sp_ref, tt_ref, et_ref, tb_ref, eb_ref,
             part_ref, out_ref):
    f32 = jnp.float32
    T = [[tt_ref[i, j] for j in range(4)] for i in range(4)]
    E = [[et_ref[i, j] for j in range(4)] for i in range(4)]

    cp = jnp.sum(part_ref[...]) * (1.0 / _NQ)

    sx = scp_ref[0:1, :]
    sy = scp_ref[1:2, :]
    sz = scp_ref[2:3, :]
    wx = T[0][0] * sx + T[0][1] * sy + T[0][2] * sz + T[0][3]
    wy = T[1][0] * sx + T[1][1] * sy + T[1][2] * sz + T[1][3]
    wz = T[2][0] * sx + T[2][1] * sy + T[2][2] * sz + T[2][3]
    dx = rc_ref[0:1, :] - wx
    dy = rc_ref[1:2, :] - wy
    dz = rc_ref[2:3, :] - wz
    d = jnp.sqrt(dx * dx + dy * dy + dz * dz)
    col = lax.broadcasted_iota(jnp.int32, d.shape, 1)
    fcnt = jnp.sum(jnp.where((d < 0.1) & (col < _NQ), 1.0, 0.0))
    f_prec = fcnt * (1.0 / _NQ)

    Tb = [[tb_ref[i, j] for j in range(4)] for i in range(4)]
    Eb = [[eb_ref[i, j] for j in range(4)] for i in range(4)]
    tr = sum(Tb[k][i] * Eb[k][i] for k in range(3) for i in range(3))
    x = jnp.clip(0.5 * (tr - 1.0), -1.0 + 1e-7, 1.0 - 1e-7)
    xa = jnp.full((8, 128), x, f32)
    a = jnp.abs(xa)
    p = (((((((-0.0012624911 * a + 0.0066700901) * a - 0.0170881256) * a
             + 0.0308918810) * a - 0.0501743046) * a + 0.0889789874) * a
          - 0.2145988016) * a + 1.5707963050)
    acv = jnp.sqrt(jnp.maximum(1.0 - a, 0.0)) * p
    acv = jnp.where(xa < 0.0, math.pi - acv, acv)
    rre_v = acv * (180.0 / math.pi)
    rte2 = ((T[0][3] - E[0][3]) ** 2 + (T[1][3] - E[1][3]) ** 2 +
            (T[2][3] - E[2][3]) ** 2)
    rte_v = jnp.sqrt(jnp.full((8, 128), rte2, f32))

    Rr = [[sum(T[k][i] * E[k][j] for k in range(3)) for j in range(3)]
          for i in range(3)]
    t_r = [sum(T[k][i] * (E[k][3] - T[k][3]) for k in range(3))
           for i in range(3)]
    px = sp_ref[0:1, :]
    py = sp_ref[1:2, :]
    pz = sp_ref[2:3, :]
    gx = Rr[0][0] * px + Rr[0][1] * py + Rr[0][2] * pz + t_r[0] - px
    gy = Rr[1][0] * px + Rr[1][1] * py + Rr[1][2] * pz + t_r[1] - py
    gz = Rr[2][0] * px + Rr[2][1] * py + Rr[2][2] * pz + t_r[2] - pz
    dn = jnp.sqrt(gx * gx + gy * gy + gz * gz)
    col2 = lax.broadcasted_iota(jnp.int32, dn.shape, 1)
    rmse = jnp.sum(jnp.where(col2 < _NSRC, dn, 0.0)) * (1.0 / _NSRC)
    recall = jnp.where(rmse < 0.2, 1.0, 0.0)

    lane = lax.broadcasted_iota(jnp.int32, (8, 128), 1)
    out = (jnp.where(lane == 0, cp, 0.0) +
           jnp.where(lane == 1, f_prec, 0.0) +
           jnp.where(lane == 2, rre_v, 0.0) +
           jnp.where(lane == 3, rte_v, 0.0) +
           jnp.where(lane == 4, rmse, 0.0) +
           jnp.where(lane == 5, recall, 0.0))
    out_ref[...] = out


def kernel(ref_points_c, src_points_c, gt_node_corr_overlaps,
           gt_node_corr_indices, ref_node_corr_indices,
           src_node_corr_indices, ref_corr_points, src_corr_points,
           src_points, transform, estimated_transform):
    i32 = jnp.int32
    f32 = jnp.float32
    ngt_p = 32 * _GT_PT
    nq_p = 32 * _Q_PT

    gr_p = jnp.pad(gt_node_corr_indices[:, 0].astype(i32), (0, ngt_p - _NGT))
    gs_p = jnp.pad(gt_node_corr_indices[:, 1].astype(i32), (0, ngt_p - _NGT))
    ov_p = jnp.pad(gt_node_corr_overlaps.astype(f32), (0, ngt_p - _NGT))
    qr_p = jnp.pad(ref_node_corr_indices.astype(i32), (0, nq_p - _NQ),
                   constant_values=4096)
    qs_p = jnp.pad(src_node_corr_indices.astype(i32), (0, nq_p - _NQ),
                   constant_values=_TRASH_G - _MAPN)

    mesh = plsc.VectorSubcoreMesh(core_axis_name="c", subcore_axis_name="s")
    map_ref = jax.new_ref(jnp.zeros((_MAPLEN,), f32))

    pl.kernel(
        _scatter_body,
        out_type=[],
        mesh=mesh,
        scratch_types=[
            pltpu.VMEM((_GT_PT,), i32),
            pltpu.VMEM((_GT_PT,), i32),
            pltpu.VMEM((_GT_PT,), f32),
            pltpu.VMEM((25, 128), i32),
            pltpu.VMEM((128,), f32),
            pltpu.SemaphoreType.DMA,
            pltpu.SemaphoreType.DMA,
            pltpu.SemaphoreType.DMA,
            pltpu.SemaphoreType.DMA,
        ],
    )(gr_p, gs_p, ov_p, map_ref)

    partials = pl.kernel(
        _gather_body,
        out_type=jax.ShapeDtypeStruct((32, 16), f32),
        mesh=mesh,
        scratch_types=[
            pltpu.VMEM((_Q_PT,), i32),
            pltpu.VMEM((_Q_PT,), i32),
            pltpu.VMEM((13, 128), i32),
            pltpu.VMEM((13, 128), f32),
            pltpu.VMEM((16,), f32),
            pltpu.SemaphoreType.DMA,
            pltpu.SemaphoreType.DMA,
        ],
    )(qr_p, qs_p, map_ref)

    nq_c = 50176
    ns_c = 30208
    rc_pad = jnp.zeros((8, nq_c), f32).at[:3, :_NQ].set(ref_corr_points.T)
    scp_pad = jnp.zeros((8, nq_c), f32).at[:3, :_NQ].set(src_corr_points.T)
    sp_pad = jnp.zeros((8, ns_c), f32).at[:3, :_NSRC].set(src_points.T)
    tf = transform.astype(f32)
    ef = estimated_transform.astype(f32)

    def _round_bf16(x):
        xi = lax.bitcast_convert_type(x, i32)
        r = (xi + 0x7FFF + ((xi >> 16) & 1)) & (-65536)
        return lax.bitcast_convert_type(r, f32)

    tb = _round_bf16(tf)
    eb = _round_bf16(ef)

    out = pl.pallas_call(
        _tc_body,
        out_shape=jax.ShapeDtypeStruct((8, 128), f32),
        in_specs=[
            pl.BlockSpec(memory_space=pltpu.VMEM),
            pl.BlockSpec(memory_space=pltpu.VMEM),
            pl.BlockSpec(memory_space=pltpu.VMEM),
            pl.BlockSpec(memory_space=pltpu.SMEM),
            pl.BlockSpec(memory_space=pltpu.SMEM),
            pl.BlockSpec(memory_space=pltpu.SMEM),
            pl.BlockSpec(memory_space=pltpu.SMEM),
            pl.BlockSpec(memory_space=pltpu.VMEM),
        ],
        out_specs=pl.BlockSpec(memory_space=pltpu.VMEM),
    )(rc_pad, scp_pad, sp_pad, tf, ef, tb, eb, partials)
    return out[0, 0:6]


# speedup vs baseline: 10.3496x; 1.0117x over previous
import math

import jax
import jax.numpy as jnp
from jax import lax
from jax.experimental import pallas as pl
from jax.experimental.pallas import tpu as pltpu
from jax.experimental.pallas import tpu_sc as plsc

_MAPN = 16777216
_MAPLEN = _MAPN + 65552
_TRASH_S = _MAPN
_TRASH_G = _MAPN + 65536 + 8
_NGT = 100000
_NQ = 50000
_NSRC = 30000
_GT_PT = 3200
_Q_PT = 1664


def _scatter_body(gr_ref, gs_ref, ov_ref, map_ref,
                  gr_v, gs_v, ov_v, sidx_v, ones_v, sem, sem2, sem3, sem4):
    i32 = jnp.int32
    f32 = jnp.float32
    c = lax.axis_index("c")
    s = lax.axis_index("s")
    w = c * 16 + s

    pltpu.sync_copy(gr_ref.at[pl.ds(w * _GT_PT, _GT_PT)], gr_v)
    pltpu.sync_copy(gs_ref.at[pl.ds(w * _GT_PT, _GT_PT)], gs_v)
    pltpu.sync_copy(ov_ref.at[pl.ds(w * _GT_PT, _GT_PT)], ov_v)

    ones16 = jnp.ones((16,), f32)
    for i in range(8):
        ones_v[pl.ds(i * 16, 16)] = ones16

    lanei = lax.iota(i32, 16)

    def _srow(r, carry):
        for j in range(8):
            off = r * 128 + j * 16
            key = gr_v[pl.ds(off, 16)] * 4096 + gs_v[pl.ds(off, 16)]
            ok = ov_v[pl.ds(off, 16)] > 0.1
            trash = _TRASH_S + w * 2048 + ((off + lanei) & 2047)
            sidx_v[r, pl.ds(j * 16, 16)] = jnp.where(ok, key, trash)
        return carry
    lax.fori_loop(0, 25, _srow, 0)

    sems = [sem, sem2, sem3, sem4]
    copies = [pltpu.async_copy(ones_v, map_ref.at[sidx_v.at[r]], sems[r % 4])
              for r in range(25)]
    for cp in copies:
        cp.wait()


def _gather_body(qr_ref, qs_ref, map_ref, part_ref,
                 qr_v, qs_v, gidx_v, gval_v, pacc_v, sem, sem2):
    i32 = jnp.int32
    f32 = jnp.float32
    c = lax.axis_index("c")
    s = lax.axis_index("s")
    w = c * 16 + s

    pltpu.sync_copy(qr_ref.at[pl.ds(w * _Q_PT, _Q_PT)], qr_v)
    pltpu.sync_copy(qs_ref.at[pl.ds(w * _Q_PT, _Q_PT)], qs_v)

    def _qrow(r, carry):
        for j in range(8):
            off = r * 128 + j * 16
            qk = qr_v[pl.ds(off, 16)] * 4096 + qs_v[pl.ds(off, 16)]
            gidx_v[r, pl.ds(j * 16, 16)] = qk
        return carry
    lax.fori_loop(0, 13, _qrow, 0)

    gsems = [sem, sem2]
    copies = [pltpu.async_copy(map_ref.at[gidx_v.at[r]], gval_v.at[r],
                               gsems[r % 2])
              for r in range(13)]
    for cp in copies:
        cp.wait()

    def _acc(r, acc):
        for j in range(8):
            acc = acc + gval_v[r, pl.ds(j * 16, 16)]
        return acc
    acc = lax.fori_loop(0, 13, _acc, jnp.zeros((16,), f32))
    pacc_v[pl.ds(0, 16)] = acc
    pltpu.sync_copy(pacc_v, part_ref.at[w])


def _tc_body(rc_ref, scp_ref, sp_ref, tt_ref, et_ref, tb_ref, eb_ref,
             out_ref):
    f32 = jnp.float32
    T = [[tt_ref[i, j] for j in range(4)] for i in range(4)]
    E = [[et_ref[i, j] for j in range(4)] for i in range(4)]

    sx = scp_ref[0:1, :]
    sy = scp_ref[1:2, :]
    sz = scp_ref[2:3, :]
    wx = T[0][0] * sx + T[0][1] * sy + T[0][2] * sz + T[0][3]
    wy = T[1][0] * sx + T[1][1] * sy + T[1][2] * sz + T[1][3]
    wz = T[2][0] * sx + T[2][1] * sy + T[2][2] * sz + T[2][3]
    dx = rc_ref[0:1, :] - wx
    dy = rc_ref[1:2, :] - wy
    dz = rc_ref[2:3, :] - wz
    d = jnp.sqrt(dx * dx + dy * dy + dz * dz)
    col = lax.broadcasted_iota(jnp.int32, d.shape, 1)
    fcnt = jnp.sum(jnp.where((d < 0.1) & (col < _NQ), 1.0, 0.0))
    f_prec = fcnt * (1.0 / _NQ)

    Tb = [[tb_ref[i, j] for j in range(4)] for i in range(4)]
    Eb = [[eb_ref[i, j] for j in range(4)] for i in range(4)]
    tr = sum(Tb[k][i] * Eb[k][i] for k in range(3) for i in range(3))
    x = jnp.clip(0.5 * (tr - 1.0), -1.0 + 1e-7, 1.0 - 1e-7)
    xa = jnp.full((8, 128), x, f32)
    a = jnp.abs(xa)
    p = (((((((-0.0012624911 * a + 0.0066700901) * a - 0.0170881256) * a
             + 0.0308918810) * a - 0.0501743046) * a + 0.0889789874) * a
          - 0.2145988016) * a + 1.5707963050)
    acv = jnp.sqrt(jnp.maximum(1.0 - a, 0.0)) * p
    acv = jnp.where(xa < 0.0, math.pi - acv, acv)
    rre_v = acv * (180.0 / math.pi)
    rte2 = ((T[0][3] - E[0][3]) ** 2 + (T[1][3] - E[1][3]) ** 2 +
            (T[2][3] - E[2][3]) ** 2)
    rte_v = jnp.sqrt(jnp.full((8, 128), rte2, f32))

    Rr = [[sum(T[k][i] * E[k][j] for k in range(3)) for j in range(3)]
          for i in range(3)]
    t_r = [sum(T[k][i] * (E[k][3] - T[k][3]) for k in range(3))
           for i in range(3)]
    px = sp_ref[0:1, :]
    py = sp_ref[1:2, :]
    pz = sp_ref[2:3, :]
    gx = Rr[0][0] * px + Rr[0][1] * py + Rr[0][2] * pz + t_r[0] - px
    gy = Rr[1][0] * px + Rr[1][1] * py + Rr[1][2] * pz + t_r[1] - py
    gz = Rr[2][0] * px + Rr[2][1] * py + Rr[2][2] * pz + t_r[2] - pz
    dn = jnp.sqrt(gx * gx + gy * gy + gz * gz)
    col2 = lax.broadcasted_iota(jnp.int32, dn.shape, 1)
    rmse = jnp.sum(jnp.where(col2 < _NSRC, dn, 0.0)) * (1.0 / _NSRC)
    recall = jnp.where(rmse < 0.2, 1.0, 0.0)

    lane = lax.broadcasted_iota(jnp.int32, (8, 128), 1)
    out = (jnp.where(lane == 1, f_prec, 0.0) +
           jnp.where(lane == 2, rre_v, 0.0) +
           jnp.where(lane == 3, rte_v, 0.0) +
           jnp.where(lane == 4, rmse, 0.0) +
           jnp.where(lane == 5, recall, 0.0))
    out_ref[...] = out


def _final_body(dense_ref, part_ref, out_ref):
    cp = jnp.sum(part_ref[...]) * (1.0 / _NQ)
    lane = lax.broadcasted_iota(jnp.int32, (8, 128), 1)
    out_ref[...] = dense_ref[...] + jnp.where(lane == 0, cp, 0.0)


def kernel(ref_points_c, src_points_c, gt_node_corr_overlaps,
           gt_node_corr_indices, ref_node_corr_indices,
           src_node_corr_indices, ref_corr_points, src_corr_points,
           src_points, transform, estimated_transform):
    i32 = jnp.int32
    f32 = jnp.float32
    ngt_p = 32 * _GT_PT
    nq_p = 32 * _Q_PT

    gr_p = jnp.pad(gt_node_corr_indices[:, 0].astype(i32), (0, ngt_p - _NGT))
    gs_p = jnp.pad(gt_node_corr_indices[:, 1].astype(i32), (0, ngt_p - _NGT))
    ov_p = jnp.pad(gt_node_corr_overlaps.astype(f32), (0, ngt_p - _NGT))
    qr_p = jnp.pad(ref_node_corr_indices.astype(i32), (0, nq_p - _NQ),
                   constant_values=4096)
    qs_p = jnp.pad(src_node_corr_indices.astype(i32), (0, nq_p - _NQ),
                   constant_values=_TRASH_G - _MAPN)

    mesh = plsc.VectorSubcoreMesh(core_axis_name="c", subcore_axis_name="s")
    map_ref = jax.new_ref(jnp.zeros((_MAPLEN,), f32))

    pl.kernel(
        _scatter_body,
        out_type=[],
        mesh=mesh,
        scratch_types=[
            pltpu.VMEM((_GT_PT,), i32),
            pltpu.VMEM((_GT_PT,), i32),
            pltpu.VMEM((_GT_PT,), f32),
            pltpu.VMEM((25, 128), i32),
            pltpu.VMEM((128,), f32),
            pltpu.SemaphoreType.DMA,
            pltpu.SemaphoreType.DMA,
            pltpu.SemaphoreType.DMA,
            pltpu.SemaphoreType.DMA,
        ],
    )(gr_p, gs_p, ov_p, map_ref)

    partials = pl.kernel(
        _gather_body,
        out_type=jax.ShapeDtypeStruct((32, 16), f32),
        mesh=mesh,
        scratch_types=[
            pltpu.VMEM((_Q_PT,), i32),
            pltpu.VMEM((_Q_PT,), i32),
            pltpu.VMEM((13, 128), i32),
            pltpu.VMEM((13, 128), f32),
            pltpu.VMEM((16,), f32),
            pltpu.SemaphoreType.DMA,
            pltpu.SemaphoreType.DMA,
        ],
    )(qr_p, qs_p, map_ref)

    nq_c = 50176
    ns_c = 30208
    rc_pad = jnp.zeros((8, nq_c), f32).at[:3, :_NQ].set(ref_corr_points.T)
    scp_pad = jnp.zeros((8, nq_c), f32).at[:3, :_NQ].set(src_corr_points.T)
    sp_pad = jnp.zeros((8, ns_c), f32).at[:3, :_NSRC].set(src_points.T)
    tf = transform.astype(f32)
    ef = estimated_transform.astype(f32)

    def _round_bf16(x):
        xi = lax.bitcast_convert_type(x, i32)
        r = (xi + 0x7FFF + ((xi >> 16) & 1)) & (-65536)
        return lax.bitcast_convert_type(r, f32)

    tb = _round_bf16(tf)
    eb = _round_bf16(ef)

    dense = pl.pallas_call(
        _tc_body,
        out_shape=jax.ShapeDtypeStruct((8, 128), f32),
        in_specs=[
            pl.BlockSpec(memory_space=pltpu.VMEM),
            pl.BlockSpec(memory_space=pltpu.VMEM),
            pl.BlockSpec(memory_space=pltpu.VMEM),
            pl.BlockSpec(memory_space=pltpu.SMEM),
            pl.BlockSpec(memory_space=pltpu.SMEM),
            pl.BlockSpec(memory_space=pltpu.SMEM),
            pl.BlockSpec(memory_space=pltpu.SMEM),
        ],
        out_specs=pl.BlockSpec(memory_space=pltpu.VMEM),
    )(rc_pad, scp_pad, sp_pad, tf, ef, tb, eb)

    out = pl.pallas_call(
        _final_body,
        out_shape=jax.ShapeDtypeStruct((8, 128), f32),
        in_specs=[
            pl.BlockSpec(memory_space=pltpu.VMEM),
            pl.BlockSpec(memory_space=pltpu.VMEM),
        ],
        out_specs=pl.BlockSpec(memory_space=pltpu.VMEM),
    )(dense, partials)
    return out[0, 0:6]


# speedup vs baseline: 11.0015x; 1.0630x over previous
import math

import jax
import jax.numpy as jnp
from jax import lax
from jax.experimental import pallas as pl
from jax.experimental.pallas import tpu as pltpu
from jax.experimental.pallas import tpu_sc as plsc

_MAPN = 16777216
_MAPLEN = _MAPN + 1048592
_TRASH_S = _MAPN
_TRASH_G = _MAPN + 1048576 + 8
_NGT = 100000
_NQ = 50000
_NSRC = 30000
_GT_PT = 3200
_Q_PT = 1664


def _scatter_body(gr_ref, gs_ref, ov_ref, map_ref,
                  gr_v, gs_v, ov_v, sidx_v, ones_v, sem, sem2, sem3, sem4):
    i32 = jnp.int32
    f32 = jnp.float32
    c = lax.axis_index("c")
    s = lax.axis_index("s")
    w = c * 16 + s

    pltpu.sync_copy(gr_ref.at[pl.ds(w * _GT_PT, _GT_PT)], gr_v)
    pltpu.sync_copy(gs_ref.at[pl.ds(w * _GT_PT, _GT_PT)], gs_v)
    pltpu.sync_copy(ov_ref.at[pl.ds(w * _GT_PT, _GT_PT)], ov_v)

    ones16 = jnp.ones((16,), f32)
    for i in range(8):
        ones_v[pl.ds(i * 16, 16)] = ones16

    lanei = lax.iota(i32, 16)

    def _srow(r, carry):
        for j in range(8):
            off = r * 128 + j * 16
            key = gr_v[pl.ds(off, 16)] * 4096 + gs_v[pl.ds(off, 16)]
            ok = ov_v[pl.ds(off, 16)] > 0.1
            trash = _TRASH_S + w * 32768 + (((off + lanei) * 16) & 32767)
            sidx_v[r, pl.ds(j * 16, 16)] = jnp.where(ok, key, trash)
        return carry
    lax.fori_loop(0, 25, _srow, 0)

    sems = [sem, sem2, sem3, sem4]
    copies = [pltpu.async_copy(ones_v, map_ref.at[sidx_v.at[r]], sems[r % 4])
              for r in range(25)]
    for cp in copies:
        cp.wait()


def _gather_body(qr_ref, qs_ref, map_ref, part_ref,
                 qr_v, qs_v, gidx_v, gval_v, pacc_v, sem, sem2):
    i32 = jnp.int32
    f32 = jnp.float32
    c = lax.axis_index("c")
    s = lax.axis_index("s")
    w = c * 16 + s

    pltpu.sync_copy(qr_ref.at[pl.ds(w * _Q_PT, _Q_PT)], qr_v)
    pltpu.sync_copy(qs_ref.at[pl.ds(w * _Q_PT, _Q_PT)], qs_v)

    def _qrow(r, carry):
        for j in range(8):
            off = r * 128 + j * 16
            qk = qr_v[pl.ds(off, 16)] * 4096 + qs_v[pl.ds(off, 16)]
            gidx_v[r, pl.ds(j * 16, 16)] = qk
        return carry
    lax.fori_loop(0, 13, _qrow, 0)

    gsems = [sem, sem2]
    copies = [pltpu.async_copy(map_ref.at[gidx_v.at[r]], gval_v.at[r],
                               gsems[r % 2])
              for r in range(13)]
    for cp in copies:
        cp.wait()

    def _acc(r, acc):
        for j in range(8):
            acc = acc + gval_v[r, pl.ds(j * 16, 16)]
        return acc
    acc = lax.fori_loop(0, 13, _acc, jnp.zeros((16,), f32))
    pacc_v[pl.ds(0, 16)] = acc
    pltpu.sync_copy(pacc_v, part_ref.at[w])


def _tc_body(rc_ref, scp_ref, sp_ref, tt_ref, et_ref, tb_ref, eb_ref,
             out_ref):
    f32 = jnp.float32
    T = [[tt_ref[i, j] for j in range(4)] for i in range(4)]
    E = [[et_ref[i, j] for j in range(4)] for i in range(4)]

    sx = scp_ref[0:1, :]
    sy = scp_ref[1:2, :]
    sz = scp_ref[2:3, :]
    wx = T[0][0] * sx + T[0][1] * sy + T[0][2] * sz + T[0][3]
    wy = T[1][0] * sx + T[1][1] * sy + T[1][2] * sz + T[1][3]
    wz = T[2][0] * sx + T[2][1] * sy + T[2][2] * sz + T[2][3]
    dx = rc_ref[0:1, :] - wx
    dy = rc_ref[1:2, :] - wy
    dz = rc_ref[2:3, :] - wz
    d = jnp.sqrt(dx * dx + dy * dy + dz * dz)
    col = lax.broadcasted_iota(jnp.int32, d.shape, 1)
    fcnt = jnp.sum(jnp.where((d < 0.1) & (col < _NQ), 1.0, 0.0))
    f_prec = fcnt * (1.0 / _NQ)

    Tb = [[tb_ref[i, j] for j in range(4)] for i in range(4)]
    Eb = [[eb_ref[i, j] for j in range(4)] for i in range(4)]
    tr = sum(Tb[k][i] * Eb[k][i] for k in range(3) for i in range(3))
    x = jnp.clip(0.5 * (tr - 1.0), -1.0 + 1e-7, 1.0 - 1e-7)
    xa = jnp.full((8, 128), x, f32)
    a = jnp.abs(xa)
    p = (((((((-0.0012624911 * a + 0.0066700901) * a - 0.0170881256) * a
             + 0.0308918810) * a - 0.0501743046) * a + 0.0889789874) * a
          - 0.2145988016) * a + 1.5707963050)
    acv = jnp.sqrt(jnp.maximum(1.0 - a, 0.0)) * p
    acv = jnp.where(xa < 0.0, math.pi - acv, acv)
    rre_v = acv * (180.0 / math.pi)
    rte2 = ((T[0][3] - E[0][3]) ** 2 + (T[1][3] - E[1][3]) ** 2 +
            (T[2][3] - E[2][3]) ** 2)
    rte_v = jnp.sqrt(jnp.full((8, 128), rte2, f32))

    Rr = [[sum(T[k][i] * E[k][j] for k in range(3)) for j in range(3)]
          for i in range(3)]
    t_r = [sum(T[k][i] * (E[k][3] - T[k][3]) for k in range(3))
           for i in range(3)]
    px = sp_ref[0:1, :]
    py = sp_ref[1:2, :]
    pz = sp_ref[2:3, :]
    gx = Rr[0][0] * px + Rr[0][1] * py + Rr[0][2] * pz + t_r[0] - px
    gy = Rr[1][0] * px + Rr[1][1] * py + Rr[1][2] * pz + t_r[1] - py
    gz = Rr[2][0] * px + Rr[2][1] * py + Rr[2][2] * pz + t_r[2] - pz
    dn = jnp.sqrt(gx * gx + gy * gy + gz * gz)
    col2 = lax.broadcasted_iota(jnp.int32, dn.shape, 1)
    rmse = jnp.sum(jnp.where(col2 < _NSRC, dn, 0.0)) * (1.0 / _NSRC)
    recall = jnp.where(rmse < 0.2, 1.0, 0.0)

    lane = lax.broadcasted_iota(jnp.int32, (8, 128), 1)
    out = (jnp.where(lane == 1, f_prec, 0.0) +
           jnp.where(lane == 2, rre_v, 0.0) +
           jnp.where(lane == 3, rte_v, 0.0) +
           jnp.where(lane == 4, rmse, 0.0) +
           jnp.where(lane == 5, recall, 0.0))
    out_ref[...] = out


def _final_body(dense_ref, part_ref, out_ref):
    cp = jnp.sum(part_ref[...]) * (1.0 / _NQ)
    lane = lax.broadcasted_iota(jnp.int32, (8, 128), 1)
    out_ref[...] = dense_ref[...] + jnp.where(lane == 0, cp, 0.0)


def kernel(ref_points_c, src_points_c, gt_node_corr_overlaps,
           gt_node_corr_indices, ref_node_corr_indices,
           src_node_corr_indices, ref_corr_points, src_corr_points,
           src_points, transform, estimated_transform):
    i32 = jnp.int32
    f32 = jnp.float32
    ngt_p = 32 * _GT_PT
    nq_p = 32 * _Q_PT

    gr_p = jnp.pad(gt_node_corr_indices[:, 0].astype(i32), (0, ngt_p - _NGT))
    gs_p = jnp.pad(gt_node_corr_indices[:, 1].astype(i32), (0, ngt_p - _NGT))
    ov_p = jnp.pad(gt_node_corr_overlaps.astype(f32), (0, ngt_p - _NGT))
    qr_p = jnp.pad(ref_node_corr_indices.astype(i32), (0, nq_p - _NQ),
                   constant_values=4096)
    qs_p = jnp.pad(src_node_corr_indices.astype(i32), (0, nq_p - _NQ),
                   constant_values=_TRASH_G - _MAPN)

    mesh = plsc.VectorSubcoreMesh(core_axis_name="c", subcore_axis_name="s")
    map_ref = jax.new_ref(jnp.zeros((_MAPLEN,), f32))

    pl.kernel(
        _scatter_body,
        out_type=[],
        mesh=mesh,
        scratch_types=[
            pltpu.VMEM((_GT_PT,), i32),
            pltpu.VMEM((_GT_PT,), i32),
            pltpu.VMEM((_GT_PT,), f32),
            pltpu.VMEM((25, 128), i32),
            pltpu.VMEM((128,), f32),
            pltpu.SemaphoreType.DMA,
            pltpu.SemaphoreType.DMA,
            pltpu.SemaphoreType.DMA,
            pltpu.SemaphoreType.DMA,
        ],
    )(gr_p, gs_p, ov_p, map_ref)

    partials = pl.kernel(
        _gather_body,
        out_type=jax.ShapeDtypeStruct((32, 16), f32),
        mesh=mesh,
        scratch_types=[
            pltpu.VMEM((_Q_PT,), i32),
            pltpu.VMEM((_Q_PT,), i32),
            pltpu.VMEM((13, 128), i32),
            pltpu.VMEM((13, 128), f32),
            pltpu.VMEM((16,), f32),
            pltpu.SemaphoreType.DMA,
            pltpu.SemaphoreType.DMA,
        ],
    )(qr_p, qs_p, map_ref)

    nq_c = 50176
    ns_c = 30208
    rc_pad = jnp.zeros((8, nq_c), f32).at[:3, :_NQ].set(ref_corr_points.T)
    scp_pad = jnp.zeros((8, nq_c), f32).at[:3, :_NQ].set(src_corr_points.T)
    sp_pad = jnp.zeros((8, ns_c), f32).at[:3, :_NSRC].set(src_points.T)
    tf = transform.astype(f32)
    ef = estimated_transform.astype(f32)

    def _round_bf16(x):
        xi = lax.bitcast_convert_type(x, i32)
        r = (xi + 0x7FFF + ((xi >> 16) & 1)) & (-65536)
        return lax.bitcast_convert_type(r, f32)

    tb = _round_bf16(tf)
    eb = _round_bf16(ef)

    dense = pl.pallas_call(
        _tc_body,
        out_shape=jax.ShapeDtypeStruct((8, 128), f32),
        in_specs=[
            pl.BlockSpec(memory_space=pltpu.VMEM),
            pl.BlockSpec(memory_space=pltpu.VMEM),
            pl.BlockSpec(memory_space=pltpu.VMEM),
            pl.BlockSpec(memory_space=pltpu.SMEM),
            pl.BlockSpec(memory_space=pltpu.SMEM),
            pl.BlockSpec(memory_space=pltpu.SMEM),
            pl.BlockSpec(memory_space=pltpu.SMEM),
        ],
        out_specs=pl.BlockSpec(memory_space=pltpu.VMEM),
    )(rc_pad, scp_pad, sp_pad, tf, ef, tb, eb)

    out = pl.pallas_call(
        _final_body,
        out_shape=jax.ShapeDtypeStruct((8, 128), f32),
        in_specs=[
            pl.BlockSpec(memory_space=pltpu.VMEM),
            pl.BlockSpec(memory_space=pltpu.VMEM),
        ],
        out_specs=pl.BlockSpec(memory_space=pltpu.VMEM),
    )(dense, partials)
    return out[0, 0:6]
